# baseline (speedup 1.0000x reference)
import functools

import jax
import jax.numpy as jnp
import numpy as np
from jax import lax
from jax.experimental import pallas as pl
from jax.experimental.pallas import tpu as pltpu
from jax.experimental.pallas import tpu_sc as plsc

_HASH_PRIMES = [2654435761, 2246822519, 3266489917, 2028178513, 1220703125,
                1610612741, 805306457, 402653189, 3674653429, 2860486313,
                1073676287, 2971215073, 1500450271, 3267000013, 2654435789,
                4049292737, 2246822531, 3266489927, 2028178519, 1220703133]
_FIB = [1, 1, 2, 3, 5, 8, 13, 21]
_MATCH_OFFSETS = (1, 2, 3, 4, 5, 6, 7, 8, 12, 16, 24, 32)
_B, _T = 32, 512
_VOCAB, _BYTE_DIM = 1024, 128
_NUM_TABLES, _BUCKETS, _EPT = 8, 32768, 16
_HIDDEN, _NUM_LAYERS, _KSZ = 512, 2, 8
_PAD = 32
_GROUPS = 4
_GB = _B // _GROUPS
_NW = 32
_WPR = _NW // _GB
_TQ = _T // _WPR


def _sc_consts():
    c = np.zeros((4, 16), np.int32)
    for lane in range(16):
        c[2, lane] = _MATCH_OFFSETS[lane] if lane < 12 else 32
        c[3, lane] = 1 if lane < 12 else 0
    return jnp.asarray(c)


def _sc_gather(tokens_pad, ht_flat, byte_embed, consts):
    mesh = plsc.VectorSubcoreMesh(core_axis_name="c", subcore_axis_name="s")
    nhc = _TQ // 128

    @functools.partial(
        pl.kernel,
        out_type=(
            jax.ShapeDtypeStruct((_GB, _T, _BYTE_DIM), jnp.float32),
            jax.ShapeDtypeStruct((_NUM_TABLES, _GB, _T, _EPT), jnp.float32),
            jax.ShapeDtypeStruct((_GB, _T, 16), jnp.float32),
        ),
        mesh=mesh,
        compiler_params=pltpu.CompilerParams(use_tc_tiling_on_sc=False,
                                             needs_layout_passes=False),
        scratch_types=[
            pltpu.VMEM((_PAD + _TQ,), jnp.int32),
            pltpu.VMEM((4, 16), jnp.int32),
            pltpu.VMEM((_NUM_TABLES * nhc, 128), jnp.int32),
            pltpu.VMEM((128, _EPT), jnp.float32),
            pltpu.VMEM((128, _BYTE_DIM), jnp.float32),
            pltpu.VMEM((_TQ, 16), jnp.float32),
            pltpu.SemaphoreType.DMA,
        ],
    )
    def k(tokp_hbm, ht_hbm, be_hbm, cst_hbm, byte_hbm, hf_hbm, m_hbm,
          tok_v, cst_v, idx_v, hbuf, bbuf, mbuf, sem):
        wid = lax.axis_index("s") * 2 + lax.axis_index("c")
        b = wid >> np.int32(_WPR.bit_length() - 1)
        q = wid & np.int32(_WPR - 1)
        t_base = q * np.int32(_TQ)

        pltpu.sync_copy(tokp_hbm.at[b, pl.ds(t_base, _PAD + _TQ)], tok_v)
        pltpu.sync_copy(cst_hbm, cst_v)

        lane = lax.iota(jnp.int32, 16)
        moff = cst_v[2, :]
        mvalid = cst_v[3, :] > 0

        for i in range(_NUM_TABLES):
            off = _FIB[i]
            prime15 = _HASH_PRIMES[(i * 3) % len(_HASH_PRIMES)] & 32767
            for j in range(nhc):
                for g in range(128 // 16):
                    t0 = j * 128 + g * 16
                    tok16 = tok_v[pl.ds(_PAD - off + t0, 16)]
                    idx16 = ((tok16 * prime15) & 32767) + i * _BUCKETS
                    idx_v[i * nhc + j, pl.ds(g * 16, 16)] = idx16

        @pl.loop(jnp.int32(0), jnp.int32(_TQ))
        def match_body(t):
            tp = t + jnp.int32(_PAD)
            cur = plsc.load_gather(tok_v, [lane * jnp.int32(0) + tp])
            prev = plsc.load_gather(tok_v, [tp - moff])
            ok = (cur == prev) & (t + t_base >= moff) & mvalid
            mbuf[t, :] = jnp.where(ok, jnp.float32(1.0), jnp.float32(0.0))
        pltpu.sync_copy(mbuf, m_hbm.at[b, pl.ds(t_base, _TQ)])

        for i in range(_NUM_TABLES):
            for j in range(nhc):
                pltpu.async_copy(
                    ht_hbm.at[idx_v.at[jnp.int32(i * nhc + j)]],
                    hbuf, sem).wait()
                pltpu.sync_copy(
                    hbuf,
                    hf_hbm.at[jnp.int32(i), b, pl.ds(t_base + j * 128, 128)])

        for j in range(_TQ // 128):
            pltpu.async_copy(
                be_hbm.at[tok_v.at[pl.ds(_PAD + j * 128, 128)]], bbuf,
                sem).wait()
            pltpu.sync_copy(bbuf, byte_hbm.at[b, pl.ds(t_base + j * 128, 128)])

    return k(tokens_pad, ht_flat, byte_embed, consts)


def _trunk_body(byte_ref, hf_ref, m_ref, cw_ref, cb_ref, wb_ref, wh_ref,
                wm_ref, ipb_ref, w1_ref, w2_ref, wo_ref, lng_ref, lnb_ref,
                hw_ref, hb_ref, out_ref):
    f32 = jnp.float32
    bf16 = jnp.bfloat16
    bf = byte_ref[0]
    hf = jnp.concatenate([hf_ref[i, 0] for i in range(_NUM_TABLES)], axis=1)
    match = m_ref[0]

    acc = hf * cw_ref[7][None, :]
    for s in range(1, _KSZ):
        shifted = jnp.concatenate(
            [jnp.zeros((s, 128), f32), hf[:_T - s]], axis=0)
        acc = acc + shifted * cw_ref[7 - s][None, :]
    hconv = acc + cb_ref[0][None, :]

    h = (jnp.dot(bf.astype(bf16), wb_ref[...], preferred_element_type=f32)
         + jnp.dot(hconv.astype(bf16), wh_ref[...], preferred_element_type=f32)
         + jnp.dot(match.astype(bf16), wm_ref[...], preferred_element_type=f32)
         + ipb_ref[0][None, :])

    for l in range(_NUM_LAYERS):
        hb16 = h.astype(bf16)
        u = jnp.dot(hb16, w1_ref[l], preferred_element_type=f32)
        v = jnp.dot(hb16, w2_ref[l], preferred_element_type=f32)
        a = (u * jax.nn.sigmoid(u)) * v
        a = jnp.dot(a.astype(bf16), wo_ref[l], preferred_element_type=f32)
        x = a + h
        m = jnp.mean(x, axis=-1, keepdims=True)
        xc = x - m
        var = jnp.mean(xc * xc, axis=-1, keepdims=True)
        h = xc * lax.rsqrt(var + 1e-5) * lng_ref[l][None, :] + lnb_ref[l][None, :]

    out_ref[0] = (jnp.dot(h.astype(bf16), hw_ref[...], preferred_element_type=f32)
                  + hb_ref[0][None, :])


def _trunk(byte_feat, hfeat, match, cw8, cb, wb, wh, wm, ipb, w1t, w2t, wot,
           ln_g, ln_b, hwt, hb, interpret=False):
    _z = lambda: jnp.int32(0)
    full = lambda *shape: pl.BlockSpec(
        shape, lambda b, _n=len(shape): tuple(_z() for _ in range(_n)))
    per_b = lambda *shape: pl.BlockSpec(
        (1,) + shape, lambda b, _n=len(shape): (b,) + tuple(_z() for _ in range(_n)))
    return pl.pallas_call(
        _trunk_body,
        grid=(_GB,),
        in_specs=[
            per_b(_T, _BYTE_DIM),
            pl.BlockSpec((_NUM_TABLES, 1, _T, _EPT),
                         lambda b: (jnp.int32(0), b, jnp.int32(0),
                                    jnp.int32(0))),
            per_b(_T, 16),
            full(_KSZ, 128),
            full(1, 128),
            full(128, _HIDDEN),
            full(128, _HIDDEN),
            full(16, _HIDDEN),
            full(1, _HIDDEN),
            full(_NUM_LAYERS, _HIDDEN, _HIDDEN),
            full(_NUM_LAYERS, _HIDDEN, _HIDDEN),
            full(_NUM_LAYERS, _HIDDEN, _HIDDEN),
            full(_NUM_LAYERS, _HIDDEN),
            full(_NUM_LAYERS, _HIDDEN),
            full(_HIDDEN, _VOCAB),
            full(1, _VOCAB),
        ],
        out_specs=per_b(_T, _VOCAB),
        out_shape=jax.ShapeDtypeStruct((_GB, _T, _VOCAB), jnp.float32),
        interpret=interpret,
    )(byte_feat, hfeat, match, cw8, cb, wb, wh, wm, ipb, w1t, w2t, wot,
      ln_g, ln_b, hwt, hb)


def kernel(tokens, byte_embed, hash_tables, conv_w, conv_b, in_proj_w,
           in_proj_b, w1, w2, wo, ln_g, ln_b, head_w, head_b):
    tokens_i32 = tokens.astype(jnp.int32)
    tokens_pad = jnp.pad(tokens_i32, ((0, 0), (_PAD, 0)))
    ht_flat = hash_tables.reshape(_NUM_TABLES * _BUCKETS, _EPT)
    consts = _sc_consts()

    bf16 = jnp.bfloat16
    cw8 = conv_w[:, 0, :].T
    wb = in_proj_w[:, :128].T.astype(bf16)
    wh = in_proj_w[:, 128:256].T.astype(bf16)
    wm = jnp.pad(in_proj_w[:, 256:268].T, ((0, 4), (0, 0))).astype(bf16)
    w1t = jnp.transpose(w1, (0, 2, 1)).astype(bf16)
    w2t = jnp.transpose(w2, (0, 2, 1)).astype(bf16)
    wot = jnp.transpose(wo, (0, 2, 1)).astype(bf16)
    hwt = head_w.T.astype(bf16)

    outs = []
    for g in range(_GROUPS):
        byte_feat, hfeat, match = _sc_gather(
            tokens_pad[g * _GB:(g + 1) * _GB], ht_flat, byte_embed, consts)
        outs.append(_trunk(byte_feat, hfeat, match, cw8, conv_b[None, :], wb,
                           wh, wm, in_proj_b[None, :], w1t, w2t, wot, ln_g,
                           ln_b, hwt, head_b[None, :]))
    return jnp.concatenate(outs, axis=0)


# speedup vs baseline: 1.1413x; 1.1413x over previous
import functools

import jax
import jax.numpy as jnp
import numpy as np
from jax import lax
from jax.experimental import pallas as pl
from jax.experimental.pallas import tpu as pltpu
from jax.experimental.pallas import tpu_sc as plsc

_HASH_PRIMES = [2654435761, 2246822519, 3266489917, 2028178513, 1220703125,
                1610612741, 805306457, 402653189, 3674653429, 2860486313,
                1073676287, 2971215073, 1500450271, 3267000013, 2654435789,
                4049292737, 2246822531, 3266489927, 2028178519, 1220703133]
_FIB = [1, 1, 2, 3, 5, 8, 13, 21]
_MATCH_OFFSETS = (1, 2, 3, 4, 5, 6, 7, 8, 12, 16, 24, 32)
_B, _T = 32, 512
_VOCAB, _BYTE_DIM = 1024, 128
_NUM_TABLES, _BUCKETS, _EPT = 8, 32768, 16
_HIDDEN, _NUM_LAYERS, _KSZ = 512, 2, 8
_PAD = 32


def _sc_consts():
    c = np.zeros((4, 16), np.int32)
    for lane in range(16):
        tb = lane & 7
        c[0, lane] = (lane >> 3) - _FIB[tb]
        c[1, lane] = _HASH_PRIMES[(tb * 3) % len(_HASH_PRIMES)] & 32767
    for lane in range(16):
        c[2, lane] = _MATCH_OFFSETS[lane] if lane < 12 else 32
        c[3, lane] = 1 if lane < 12 else 0
    return jnp.asarray(c)


def _sc_gather(tokens_pad, ht_flat, byte_embed, consts):
    mesh = plsc.VectorSubcoreMesh(core_axis_name="c", subcore_axis_name="s")
    n_chunks = _T // 16

    @functools.partial(
        pl.kernel,
        out_type=(
            jax.ShapeDtypeStruct((_B, _T, _BYTE_DIM), jnp.float32),
            jax.ShapeDtypeStruct((_NUM_TABLES, _B, _T, _EPT), jnp.float32),
            jax.ShapeDtypeStruct((_B, _T, 16), jnp.float32),
        ),
        mesh=mesh,
        compiler_params=pltpu.CompilerParams(use_tc_tiling_on_sc=False,
                                             needs_layout_passes=False),
        scratch_types=[
            pltpu.VMEM((_PAD + _T,), jnp.int32),
            pltpu.VMEM((4, 16), jnp.int32),
            pltpu.VMEM((n_chunks, 128), jnp.int32),
            pltpu.VMEM((128, _EPT), jnp.float32),
            pltpu.VMEM((128, _BYTE_DIM), jnp.float32),
            pltpu.VMEM((_T, 16), jnp.float32),
            pltpu.SemaphoreType.DMA,
        ],
    )
    def k(tokp_hbm, ht_hbm, be_hbm, cst_hbm, byte_hbm, hf_hbm, m_hbm,
          tok_v, cst_v, idx_v, hbuf, bbuf, mbuf, sem):
        wid = lax.axis_index("s") * 2 + lax.axis_index("c")
        b = wid

        pltpu.sync_copy(tokp_hbm.at[b], tok_v)
        pltpu.sync_copy(cst_hbm, cst_v)

        lane = lax.iota(jnp.int32, 16)
        moff = cst_v[2, :]
        mvalid = cst_v[3, :] > 0

        for i in range(_NUM_TABLES):
            off = _FIB[i]
            prime15 = _HASH_PRIMES[(i * 3) % len(_HASH_PRIMES)] & 32767
            for j in range(_T // 128):
                for g in range(128 // 16):
                    t0 = j * 128 + g * 16
                    tok16 = tok_v[pl.ds(_PAD - off + t0, 16)]
                    idx16 = ((tok16 * prime15) & 32767) + i * _BUCKETS
                    idx_v[i * (_T // 128) + j, pl.ds(g * 16, 16)] = idx16

        @pl.loop(jnp.int32(0), jnp.int32(_T))
        def match_body(t):
            tp = t + jnp.int32(_PAD)
            cur = plsc.load_gather(tok_v, [lane * jnp.int32(0) + tp])
            prev = plsc.load_gather(tok_v, [tp - moff])
            ok = (cur == prev) & (t >= moff) & mvalid
            mbuf[t, :] = jnp.where(ok, jnp.float32(1.0), jnp.float32(0.0))
        pltpu.sync_copy(mbuf, m_hbm.at[b])

        for i in range(_NUM_TABLES):
            for j in range(_T // 128):
                pltpu.async_copy(
                    ht_hbm.at[idx_v.at[jnp.int32(i * (_T // 128) + j)]],
                    hbuf, sem).wait()
                pltpu.sync_copy(
                    hbuf, hf_hbm.at[jnp.int32(i), b, pl.ds(j * 128, 128)])

        for j in range(4):
            pltpu.async_copy(
                be_hbm.at[tok_v.at[pl.ds(_PAD + j * 128, 128)]], bbuf,
                sem).wait()
            pltpu.sync_copy(bbuf, byte_hbm.at[b, pl.ds(j * 128, 128)])

    return k(tokens_pad, ht_flat, byte_embed, consts)


def _trunk_body(byte_ref, hf_ref, m_ref, cw_ref, cb_ref, wb_ref, wh_ref,
                wm_ref, ipb_ref, w1_ref, w2_ref, wo_ref, lng_ref, lnb_ref,
                hw_ref, hb_ref, out_ref):
    f32 = jnp.float32
    bf16 = jnp.bfloat16
    bf = byte_ref[0]
    hf = jnp.concatenate([hf_ref[i, 0] for i in range(_NUM_TABLES)], axis=1)
    match = m_ref[0]

    acc = hf * cw_ref[7][None, :]
    for s in range(1, _KSZ):
        shifted = jnp.concatenate(
            [jnp.zeros((s, 128), f32), hf[:_T - s]], axis=0)
        acc = acc + shifted * cw_ref[7 - s][None, :]
    hconv = acc + cb_ref[0][None, :]

    h = (jnp.dot(bf.astype(bf16), wb_ref[...], preferred_element_type=f32)
         + jnp.dot(hconv.astype(bf16), wh_ref[...], preferred_element_type=f32)
         + jnp.dot(match.astype(bf16), wm_ref[...], preferred_element_type=f32)
         + ipb_ref[0][None, :])

    for l in range(_NUM_LAYERS):
        hb16 = h.astype(bf16)
        u = jnp.dot(hb16, w1_ref[l], preferred_element_type=f32)
        v = jnp.dot(hb16, w2_ref[l], preferred_element_type=f32)
        a = (u * jax.nn.sigmoid(u)) * v
        a = jnp.dot(a.astype(bf16), wo_ref[l], preferred_element_type=f32)
        x = a + h
        m = jnp.mean(x, axis=-1, keepdims=True)
        xc = x - m
        var = jnp.mean(xc * xc, axis=-1, keepdims=True)
        h = xc * lax.rsqrt(var + 1e-5) * lng_ref[l][None, :] + lnb_ref[l][None, :]

    out_ref[0] = (jnp.dot(h.astype(bf16), hw_ref[...], preferred_element_type=f32)
                  + hb_ref[0][None, :])


def _trunk(byte_feat, hfeat, match, cw8, cb, wb, wh, wm, ipb, w1t, w2t, wot,
           ln_g, ln_b, hwt, hb, interpret=False):
    _z = lambda: jnp.int32(0)
    full = lambda *shape: pl.BlockSpec(
        shape, lambda b, _n=len(shape): tuple(_z() for _ in range(_n)))
    per_b = lambda *shape: pl.BlockSpec(
        (1,) + shape, lambda b, _n=len(shape): (b,) + tuple(_z() for _ in range(_n)))
    return pl.pallas_call(
        _trunk_body,
        grid=(_B,),
        in_specs=[
            per_b(_T, _BYTE_DIM),
            pl.BlockSpec((_NUM_TABLES, 1, _T, _EPT),
                         lambda b: (jnp.int32(0), b, jnp.int32(0),
                                    jnp.int32(0))),
            per_b(_T, 16),
            full(_KSZ, 128),
            full(1, 128),
            full(128, _HIDDEN),
            full(128, _HIDDEN),
            full(16, _HIDDEN),
            full(1, _HIDDEN),
            full(_NUM_LAYERS, _HIDDEN, _HIDDEN),
            full(_NUM_LAYERS, _HIDDEN, _HIDDEN),
            full(_NUM_LAYERS, _HIDDEN, _HIDDEN),
            full(_NUM_LAYERS, _HIDDEN),
            full(_NUM_LAYERS, _HIDDEN),
            full(_HIDDEN, _VOCAB),
            full(1, _VOCAB),
        ],
        out_specs=per_b(_T, _VOCAB),
        out_shape=jax.ShapeDtypeStruct((_B, _T, _VOCAB), jnp.float32),
        interpret=interpret,
    )(byte_feat, hfeat, match, cw8, cb, wb, wh, wm, ipb, w1t, w2t, wot,
      ln_g, ln_b, hwt, hb)


def kernel(tokens, byte_embed, hash_tables, conv_w, conv_b, in_proj_w,
           in_proj_b, w1, w2, wo, ln_g, ln_b, head_w, head_b):
    tokens_i32 = tokens.astype(jnp.int32)
    tokens_pad = jnp.pad(tokens_i32, ((0, 0), (_PAD, 0)))
    ht_flat = hash_tables.reshape(_NUM_TABLES * _BUCKETS, _EPT)

    byte_feat, hfeat, match = _sc_gather(tokens_pad, ht_flat, byte_embed,
                                         _sc_consts())

    bf16 = jnp.bfloat16
    cw8 = conv_w[:, 0, :].T
    wb = in_proj_w[:, :128].T.astype(bf16)
    wh = in_proj_w[:, 128:256].T.astype(bf16)
    wm = jnp.pad(in_proj_w[:, 256:268].T, ((0, 4), (0, 0))).astype(bf16)
    w1t = jnp.transpose(w1, (0, 2, 1)).astype(bf16)
    w2t = jnp.transpose(w2, (0, 2, 1)).astype(bf16)
    wot = jnp.transpose(wo, (0, 2, 1)).astype(bf16)
    return _trunk(byte_feat, hfeat, match, cw8, conv_b[None, :], wb, wh, wm,
                  in_proj_b[None, :], w1t, w2t, wot, ln_g, ln_b,
                  head_w.T.astype(bf16), head_b[None, :])


# speedup vs baseline: 1.1875x; 1.0405x over previous
import functools

import jax
import jax.numpy as jnp
import numpy as np
from jax import lax
from jax.experimental import pallas as pl
from jax.experimental.pallas import tpu as pltpu
from jax.experimental.pallas import tpu_sc as plsc

_HASH_PRIMES = [2654435761, 2246822519, 3266489917, 2028178513, 1220703125,
                1610612741, 805306457, 402653189, 3674653429, 2860486313,
                1073676287, 2971215073, 1500450271, 3267000013, 2654435789,
                4049292737, 2246822531, 3266489927, 2028178519, 1220703133]
_FIB = [1, 1, 2, 3, 5, 8, 13, 21]
_MATCH_OFFSETS = (1, 2, 3, 4, 5, 6, 7, 8, 12, 16, 24, 32)
_B, _T = 32, 512
_VOCAB, _BYTE_DIM = 1024, 128
_NUM_TABLES, _BUCKETS, _EPT = 8, 32768, 16
_HIDDEN, _NUM_LAYERS, _KSZ = 512, 2, 8
_PAD = 32


def _sc_consts():
    c = np.zeros((4, 16), np.int32)
    for lane in range(16):
        tb = lane & 7
        c[0, lane] = (lane >> 3) - _FIB[tb]
        c[1, lane] = _HASH_PRIMES[(tb * 3) % len(_HASH_PRIMES)] & 32767
    for lane in range(16):
        c[2, lane] = _MATCH_OFFSETS[lane] if lane < 12 else 32
        c[3, lane] = 1 if lane < 12 else 0
    return jnp.asarray(c)


def _sc_gather(tokens_pad, ht_flat, byte_embed, consts):
    mesh = plsc.VectorSubcoreMesh(core_axis_name="c", subcore_axis_name="s")
    n_chunks = _T // 16

    @functools.partial(
        pl.kernel,
        out_type=(
            jax.ShapeDtypeStruct((_B, _T, _BYTE_DIM), jnp.float32),
            jax.ShapeDtypeStruct((_NUM_TABLES, _B, _T, _EPT), jnp.float32),
            jax.ShapeDtypeStruct((_B, _T, 16), jnp.float32),
        ),
        mesh=mesh,
        compiler_params=pltpu.CompilerParams(use_tc_tiling_on_sc=False,
                                             needs_layout_passes=False),
        scratch_types=[
            pltpu.VMEM((_PAD + _T,), jnp.int32),
            pltpu.VMEM((4, 16), jnp.int32),
            pltpu.VMEM((n_chunks, 128), jnp.int32),
            pltpu.VMEM((128, _EPT), jnp.float32),
            pltpu.VMEM((128, _EPT), jnp.float32),
            pltpu.VMEM((128, _BYTE_DIM), jnp.float32),
            pltpu.VMEM((128, _BYTE_DIM), jnp.float32),
            pltpu.VMEM((_T, 16), jnp.float32),
            pltpu.SemaphoreType.DMA,
            pltpu.SemaphoreType.DMA,
        ],
    )
    def k(tokp_hbm, ht_hbm, be_hbm, cst_hbm, byte_hbm, hf_hbm, m_hbm,
          tok_v, cst_v, idx_v, hbuf0, hbuf1, bbuf0, bbuf1, mbuf, sem0, sem1):
        wid = lax.axis_index("s") * 2 + lax.axis_index("c")
        b = wid

        pltpu.sync_copy(tokp_hbm.at[b], tok_v)
        pltpu.sync_copy(cst_hbm, cst_v)

        lane = lax.iota(jnp.int32, 16)
        moff = cst_v[2, :]
        mvalid = cst_v[3, :] > 0

        for i in range(_NUM_TABLES):
            off = _FIB[i]
            prime15 = _HASH_PRIMES[(i * 3) % len(_HASH_PRIMES)] & 32767
            for j in range(_T // 128):
                for g in range(128 // 16):
                    t0 = j * 128 + g * 16
                    tok16 = tok_v[pl.ds(_PAD - off + t0, 16)]
                    idx16 = ((tok16 * prime15) & 32767) + i * _BUCKETS
                    idx_v[i * (_T // 128) + j, pl.ds(g * 16, 16)] = idx16

        @pl.loop(jnp.int32(0), jnp.int32(_T))
        def match_body(t):
            tp = t + jnp.int32(_PAD)
            cur = plsc.load_gather(tok_v, [lane * jnp.int32(0) + tp])
            prev = plsc.load_gather(tok_v, [tp - moff])
            ok = (cur == prev) & (t >= moff) & mvalid
            mbuf[t, :] = jnp.where(ok, jnp.float32(1.0), jnp.float32(0.0))
        pltpu.sync_copy(mbuf, m_hbm.at[b])

        nh = _NUM_TABLES * (_T // 128)
        hbufs, sems = (hbuf0, hbuf1), (sem0, sem1)

        def h_start(r):
            return pltpu.async_copy(
                ht_hbm.at[idx_v.at[jnp.int32(r)]], hbufs[r % 2], sems[r % 2])

        def h_dst(r):
            i, j = divmod(r, _T // 128)
            return hf_hbm.at[jnp.int32(i), b, pl.ds(j * 128, 128)]

        handles = {0: h_start(0)}
        for r in range(nh):
            if r + 1 < nh:
                handles[r + 1] = h_start(r + 1)
            handles.pop(r).wait()
            pltpu.sync_copy(hbufs[r % 2], h_dst(r))

        bbufs = (bbuf0, bbuf1)

        def b_start(j):
            return pltpu.async_copy(
                be_hbm.at[tok_v.at[pl.ds(_PAD + j * 128, 128)]],
                bbufs[j % 2], sems[j % 2])

        bh = {0: b_start(0)}
        for j in range(4):
            if j + 1 < 4:
                bh[j + 1] = b_start(j + 1)
            bh.pop(j).wait()
            pltpu.sync_copy(bbufs[j % 2], byte_hbm.at[b, pl.ds(j * 128, 128)])

    return k(tokens_pad, ht_flat, byte_embed, consts)


def _trunk_body(byte_ref, hf_ref, m_ref, cw_ref, cb_ref, wb_ref, wh_ref,
                wm_ref, ipb_ref, w1_ref, w2_ref, wo_ref, lng_ref, lnb_ref,
                hw_ref, hb_ref, out_ref):
    f32 = jnp.float32
    bf16 = jnp.bfloat16
    bf = byte_ref[0]
    hf = jnp.concatenate([hf_ref[i, 0] for i in range(_NUM_TABLES)], axis=1)
    match = m_ref[0]

    acc = hf * cw_ref[7][None, :]
    for s in range(1, _KSZ):
        shifted = jnp.concatenate(
            [jnp.zeros((s, 128), f32), hf[:_T - s]], axis=0)
        acc = acc + shifted * cw_ref[7 - s][None, :]
    hconv = acc + cb_ref[0][None, :]

    h = (jnp.dot(bf.astype(bf16), wb_ref[...], preferred_element_type=f32)
         + jnp.dot(hconv.astype(bf16), wh_ref[...], preferred_element_type=f32)
         + jnp.dot(match.astype(bf16), wm_ref[...], preferred_element_type=f32)
         + ipb_ref[0][None, :])

    for l in range(_NUM_LAYERS):
        hb16 = h.astype(bf16)
        u = jnp.dot(hb16, w1_ref[l], preferred_element_type=f32)
        v = jnp.dot(hb16, w2_ref[l], preferred_element_type=f32)
        a = (u * jax.nn.sigmoid(u)) * v
        a = jnp.dot(a.astype(bf16), wo_ref[l], preferred_element_type=f32)
        x = a + h
        m = jnp.mean(x, axis=-1, keepdims=True)
        xc = x - m
        var = jnp.mean(xc * xc, axis=-1, keepdims=True)
        h = xc * lax.rsqrt(var + 1e-5) * lng_ref[l][None, :] + lnb_ref[l][None, :]

    out_ref[0] = (jnp.dot(h.astype(bf16), hw_ref[...], preferred_element_type=f32)
                  + hb_ref[0][None, :])


def _trunk(byte_feat, hfeat, match, cw8, cb, wb, wh, wm, ipb, w1t, w2t, wot,
           ln_g, ln_b, hwt, hb, interpret=False):
    _z = lambda: jnp.int32(0)
    full = lambda *shape: pl.BlockSpec(
        shape, lambda b, _n=len(shape): tuple(_z() for _ in range(_n)))
    per_b = lambda *shape: pl.BlockSpec(
        (1,) + shape, lambda b, _n=len(shape): (b,) + tuple(_z() for _ in range(_n)))
    return pl.pallas_call(
        _trunk_body,
        grid=(_B,),
        in_specs=[
            per_b(_T, _BYTE_DIM),
            pl.BlockSpec((_NUM_TABLES, 1, _T, _EPT),
                         lambda b: (jnp.int32(0), b, jnp.int32(0),
                                    jnp.int32(0))),
            per_b(_T, 16),
            full(_KSZ, 128),
            full(1, 128),
            full(128, _HIDDEN),
            full(128, _HIDDEN),
            full(16, _HIDDEN),
            full(1, _HIDDEN),
            full(_NUM_LAYERS, _HIDDEN, _HIDDEN),
            full(_NUM_LAYERS, _HIDDEN, _HIDDEN),
            full(_NUM_LAYERS, _HIDDEN, _HIDDEN),
            full(_NUM_LAYERS, _HIDDEN),
            full(_NUM_LAYERS, _HIDDEN),
            full(_HIDDEN, _VOCAB),
            full(1, _VOCAB),
        ],
        out_specs=per_b(_T, _VOCAB),
        out_shape=jax.ShapeDtypeStruct((_B, _T, _VOCAB), jnp.float32),
        interpret=interpret,
    )(byte_feat, hfeat, match, cw8, cb, wb, wh, wm, ipb, w1t, w2t, wot,
      ln_g, ln_b, hwt, hb)


def kernel(tokens, byte_embed, hash_tables, conv_w, conv_b, in_proj_w,
           in_proj_b, w1, w2, wo, ln_g, ln_b, head_w, head_b):
    tokens_i32 = tokens.astype(jnp.int32)
    tokens_pad = jnp.pad(tokens_i32, ((0, 0), (_PAD, 0)))
    ht_flat = hash_tables.reshape(_NUM_TABLES * _BUCKETS, _EPT)

    byte_feat, hfeat, match = _sc_gather(tokens_pad, ht_flat, byte_embed,
                                         _sc_consts())

    bf16 = jnp.bfloat16
    cw8 = conv_w[:, 0, :].T
    wb = in_proj_w[:, :128].T.astype(bf16)
    wh = in_proj_w[:, 128:256].T.astype(bf16)
    wm = jnp.pad(in_proj_w[:, 256:268].T, ((0, 4), (0, 0))).astype(bf16)
    w1t = jnp.transpose(w1, (0, 2, 1)).astype(bf16)
    w2t = jnp.transpose(w2, (0, 2, 1)).astype(bf16)
    wot = jnp.transpose(wo, (0, 2, 1)).astype(bf16)
    return _trunk(byte_feat, hfeat, match, cw8, conv_b[None, :], wb, wh, wm,
                  in_proj_b[None, :], w1t, w2t, wot, ln_g, ln_b,
                  head_w.T.astype(bf16), head_b[None, :])
